# Initial kernel scaffold; baseline (speedup 1.0000x reference)
import jax
import jax.numpy as jnp
from jax.experimental import pallas as pl


def kernel(pcld_input, kpts_pre_input, cpt_pre_input, seg_pre_input):
    raise NotImplementedError("write your pallas kernel here")


# speedup vs baseline: 10.6739x; 10.6739x over previous
import jax
import jax.numpy as jnp
from jax import lax
from jax.experimental import pallas as pl

_K = 10
_NKP = 9


def _pose_kernel(offx_ref, offy_ref, offz_ref, px_ref, py_ref, pz_ref,
                 s0_ref, s1_ref, out_ref):
    offx = offx_ref[0]
    offy = offy_ref[0]
    offz = offz_ref[0]
    norm2 = offx * offx + offy * offy + offz * offz
    obj = s1_ref[0] > s0_ref[0]
    vals = jnp.where(obj, norm2, jnp.float32(1e18))

    cx = px_ref[0] + offx
    cy = py_ref[0] + offy
    cz = pz_ref[0] + offz

    n = vals.shape[1]
    iota = lax.broadcasted_iota(jnp.int32, vals.shape, 1)
    big = jnp.float32(3.0e38)
    selx, sely, selz = [], [], []
    for _ in range(_K):
        m = jnp.min(vals, axis=1, keepdims=True)
        is_m = vals == m
        idx = jnp.min(jnp.where(is_m, iota, jnp.int32(n)), axis=1,
                      keepdims=True)
        one = iota == idx
        selx.append(jnp.sum(jnp.where(one, cx, 0.0), axis=1, keepdims=True))
        sely.append(jnp.sum(jnp.where(one, cy, 0.0), axis=1, keepdims=True))
        selz.append(jnp.sum(jnp.where(one, cz, 0.0), axis=1, keepdims=True))
        vals = jnp.where(one, big, vals)

    x = jnp.concatenate(selx, axis=1)
    y = jnp.concatenate(sely, axis=1)
    z = jnp.concatenate(selz, axis=1)

    inv_k = jnp.float32(1.0 / _K)
    mx = jnp.sum(x, axis=1, keepdims=True) * inv_k
    my = jnp.sum(y, axis=1, keepdims=True) * inv_k
    mz = jnp.sum(z, axis=1, keepdims=True) * inv_k
    dx = x - mx
    dy = y - my
    dz = z - mz
    sdx = jnp.sqrt(jnp.sum(dx * dx, axis=1, keepdims=True) * inv_k)
    sdy = jnp.sqrt(jnp.sum(dy * dy, axis=1, keepdims=True) * inv_k)
    sdz = jnp.sqrt(jnp.sum(dz * dz, axis=1, keepdims=True) * inv_k)
    eps = jnp.float32(1e-9)
    inl = ((jnp.abs(dx) <= sdx + eps) & (jnp.abs(dy) <= sdy + eps)
           & (jnp.abs(dz) <= sdz + eps))
    w = inl.astype(jnp.float32)
    denom = jnp.sum(w, axis=1, keepdims=True) + jnp.float32(1e-8)
    ox = jnp.sum(x * w, axis=1, keepdims=True) / denom
    oy = jnp.sum(y * w, axis=1, keepdims=True) / denom
    oz = jnp.sum(z * w, axis=1, keepdims=True) / denom
    out_ref[0] = jnp.concatenate([ox, oy, oz], axis=1)


def kernel(pcld_input, kpts_pre_input, cpt_pre_input, seg_pre_input):
    b, n = pcld_input.shape[0], pcld_input.shape[1]
    offs = jnp.concatenate([kpts_pre_input, cpt_pre_input], axis=2)
    off_t = jnp.transpose(offs, (0, 2, 3, 1))
    offx = off_t[:, :, 0, :]
    offy = off_t[:, :, 1, :]
    offz = off_t[:, :, 2, :]
    p_t = jnp.transpose(pcld_input, (0, 2, 1))
    px = p_t[:, 0:1, :]
    py = p_t[:, 1:2, :]
    pz = p_t[:, 2:3, :]
    s_t = jnp.transpose(seg_pre_input, (0, 2, 1))
    s0 = s_t[:, 0:1, :]
    s1 = s_t[:, 1:2, :]

    wide = pl.BlockSpec((1, _NKP, n), lambda i: (i, 0, 0))
    slim = pl.BlockSpec((1, 1, n), lambda i: (i, 0, 0))
    return pl.pallas_call(
        _pose_kernel,
        grid=(b,),
        in_specs=[wide, wide, wide, slim, slim, slim, slim, slim],
        out_specs=pl.BlockSpec((1, _NKP, 3), lambda i: (i, 0, 0)),
        out_shape=jax.ShapeDtypeStruct((b, _NKP, 3), jnp.float32),
    )(offx, offy, offz, px, py, pz, s0, s1)
